# baseline (speedup 1.0000x reference)
import functools

import jax
import jax.numpy as jnp
from jax import lax
from jax.experimental import pallas as pl
from jax.experimental.pallas import tpu as pltpu
from jax.experimental.pallas import tpu_sc as plsc

B = 8
N = 2048
FDIM = 64
HID = 32
K = 6
ROWS = 1024
HALVES = 1
BH = B // HALVES

NC = 2
NS = 16
NW = NC * NS
QPW = (BH * N) // NW
CHUNK = 128
NCHUNK = QPW // CHUNK


def _knn_body(inp_ref, pc_ref, w1_ref, b1_ref, knn_ref, g_ref, *, half):
    b = pl.program_id(0)
    rt = pl.program_id(1)
    x = inp_ref[0]
    c = pc_ref[0]
    qx = x[:, 0:1]
    qy = x[:, 1:2]
    dx = qx - c[0:1, :]
    dy = qy - c[1:2, :]
    dsq = dx * dx + dy * dy
    colio = lax.broadcasted_iota(jnp.int32, (ROWS, N), 1)
    rowio = lax.broadcasted_iota(jnp.int32, (ROWS, N), 0) + rt * ROWS
    colf = colio.astype(jnp.float32)
    big = jnp.float32(jnp.inf)
    bigf = jnp.float32(float(N))
    dsq = jnp.where(colio == rowio, big, dsq)
    base = b * N
    cols = []
    for _ in range(K):
        m = jnp.min(dsq, axis=1, keepdims=True)
        cand = jnp.where(dsq == m, colf, bigf)
        am = jnp.min(cand, axis=1, keepdims=True)
        cols.append(am)
        dsq = jnp.where(cand == am, big, dsq)
    knn_ref[0] = jnp.concatenate(cols, axis=1).astype(jnp.int32) + base
    g_ref[0] = (
        jnp.dot(x[:, 2:], w1_ref[...], preferred_element_type=jnp.float32)
        + b1_ref[...]
    )


def _knn_call(inputs, pc, w1, b1, half):
    return pl.pallas_call(
        functools.partial(_knn_body, half=half),
        grid=(BH, N // ROWS),
        in_specs=[
            pl.BlockSpec((1, ROWS, 2 + FDIM), lambda b, r: (b + half * BH, r, 0)),
            pl.BlockSpec((1, 2, N), lambda b, r: (b + half * BH, 0, 0)),
            pl.BlockSpec((FDIM, HID), lambda b, r: (0, 0)),
            pl.BlockSpec((1, HID), lambda b, r: (0, 0)),
        ],
        out_specs=[
            pl.BlockSpec((1, ROWS, K), lambda b, r: (b, r, 0)),
            pl.BlockSpec((1, ROWS, HID), lambda b, r: (b, r, 0)),
        ],
        out_shape=[
            jax.ShapeDtypeStruct((BH, N, K), jnp.int32),
            jax.ShapeDtypeStruct((BH, N, HID), jnp.float32),
        ],
    )(inputs, pc, w1, b1)


def _make_gather_mean():
    mesh = plsc.VectorSubcoreMesh(core_axis_name="c", subcore_axis_name="s")

    @functools.partial(
        pl.kernel,
        mesh=mesh,
        compiler_params=pltpu.CompilerParams(use_tc_tiling_on_sc=False),
        out_type=jax.ShapeDtypeStruct((BH * N, HID), jnp.float32),
        scratch_types=[
            pltpu.VMEM((K * CHUNK,), jnp.int32),
            pltpu.VMEM((K * CHUNK,), jnp.int32),
            pltpu.VMEM((K * CHUNK, HID), jnp.float32),
            pltpu.VMEM((K * CHUNK, HID), jnp.float32),
            pltpu.VMEM((CHUNK, HID), jnp.float32),
            pltpu.VMEM((CHUNK, HID), jnp.float32),
            pltpu.SemaphoreType.DMA,
            pltpu.SemaphoreType.DMA,
            pltpu.SemaphoreType.DMA,
            pltpu.SemaphoreType.DMA,
            pltpu.SemaphoreType.DMA,
        ],
    )
    def _gather_mean(idx_hbm, g_hbm, out_hbm, idx0, idx1, rows0, rows1,
                     acc0, acc1, semi0, semi1, semg0, semg1, semo):
        wid = lax.axis_index("s") * NC + lax.axis_index("c")
        idxs = (idx0, idx1)
        rows = (rows0, rows1)
        accs = (acc0, acc1)
        semis = (semi0, semi1)
        semgs = (semg0, semg1)

        def _qbase(c):
            return wid * QPW + c * CHUNK

        idx_cp = [None] * NCHUNK
        for c in range(min(2, NCHUNK)):
            idx_cp[c] = pltpu.async_copy(
                idx_hbm.at[pl.ds(_qbase(c) * K, K * CHUNK)], idxs[c % 2], semis[c % 2]
            )

        def _fire_gathers(c):
            idx_cp[c].wait()
            return [
                pltpu.async_copy(
                    g_hbm.at[idxs[c % 2].at[pl.ds(j * CHUNK, CHUNK)]],
                    rows[c % 2].at[pl.ds(j * CHUNK, CHUNK)],
                    semgs[c % 2],
                )
                for j in range(K)
            ]

        def _accumulate(c):
            def _addq(q):
                r = q * K
                for v in range(HID // 16):
                    sl = pl.ds(v * 16, 16)
                    accs[c % 2][q, sl] = (
                        (rows[c % 2][r, sl] + rows[c % 2][r + 1, sl])
                        + (rows[c % 2][r + 2, sl] + rows[c % 2][r + 3, sl])
                        + (rows[c % 2][r + 4, sl] + rows[c % 2][r + 5, sl])
                    )

            plsc.parallel_loop(0, CHUNK, 1, unroll=8)(_addq)

        g_cp = [None] * NCHUNK
        g_cp[0] = _fire_gathers(0)
        if NCHUNK > 1:
            g_cp[1] = _fire_gathers(1)
        out_cp = [None] * NCHUNK
        for c in range(NCHUNK):
            for cp in g_cp[c]:
                cp.wait()
            if c + 2 < NCHUNK:
                idx_cp[c + 2] = pltpu.async_copy(
                    idx_hbm.at[pl.ds(_qbase(c + 2) * K, K * CHUNK)],
                    idxs[c % 2],
                    semis[c % 2],
                )
            if c >= 2:
                out_cp[c - 2].wait()
            _accumulate(c)
            out_cp[c] = pltpu.async_copy(
                accs[c % 2], out_hbm.at[pl.ds(_qbase(c), CHUNK)], semo
            )
            if c + 2 < NCHUNK:
                g_cp[c + 2] = _fire_gathers(c + 2)
        for c in range(max(0, NCHUNK - 2), NCHUNK):
            out_cp[c].wait()

    return _gather_mean


def _mlp_body(agg_ref, w2_ref, b2_ref, out_ref):
    h = jnp.maximum(agg_ref[...] * jnp.float32(1.0 / 6.0), 0.0)
    out_ref[...] = (
        jnp.dot(h, w2_ref[...], preferred_element_type=jnp.float32) + b2_ref[...]
    )


def _mlp_call(agg, w2, b2):
    return pl.pallas_call(
        _mlp_body,
        grid=((BH * N) // ROWS,),
        in_specs=[
            pl.BlockSpec((ROWS, HID), lambda r: (r, 0)),
            pl.BlockSpec((HID, FDIM), lambda r: (0, 0)),
            pl.BlockSpec((1, FDIM), lambda r: (0, 0)),
        ],
        out_specs=pl.BlockSpec((ROWS, FDIM), lambda r: (r, 0)),
        out_shape=jax.ShapeDtypeStruct((BH * N, FDIM), jnp.float32),
    )(agg, w2, b2)


def kernel(inputs, W1, b1, W2, b2):
    pos = inputs[..., :2]
    pc = jnp.transpose(pos, (0, 2, 1))
    b1r = b1.reshape(1, HID)
    b2r = b2.reshape(1, FDIM)
    gather_mean = _make_gather_mean()
    upds = []
    for half in range(HALVES):
        knn, g = _knn_call(inputs, pc, W1, b1r, half)
        agg = gather_mean(knn.reshape(BH * N * K), g.reshape(BH * N, HID))
        upds.append(_mlp_call(agg, W2, b2r).reshape(BH, N, FDIM))
    upd = jnp.concatenate(upds, axis=0)
    return jnp.concatenate([pos, upd], axis=-1)


# speedup vs baseline: 1.0033x; 1.0033x over previous
import functools

import jax
import jax.numpy as jnp
from jax import lax
from jax.experimental import pallas as pl
from jax.experimental.pallas import tpu as pltpu
from jax.experimental.pallas import tpu_sc as plsc

B = 8
N = 2048
FDIM = 64
HID = 32
K = 6
ROWS = 1024
HALVES = 1
BH = B // HALVES

NC = 2
NS = 16
NW = NC * NS
QPW = (BH * N) // NW
CHUNK = 128
NCHUNK = QPW // CHUNK


def _knn_body(inp_ref, pc_ref, w1_ref, b1_ref, knn_ref, g_ref, *, half):
    b = pl.program_id(0)
    rt = pl.program_id(1)
    x = inp_ref[0]
    c = pc_ref[0]
    qx = x[:, 0:1]
    qy = x[:, 1:2]
    dx = qx - c[0:1, :]
    dy = qy - c[1:2, :]
    dsq = dx * dx + dy * dy
    colio = lax.broadcasted_iota(jnp.int32, (ROWS, N), 1)
    rowio = lax.broadcasted_iota(jnp.int32, (ROWS, N), 0) + rt * ROWS
    colf = colio.astype(jnp.float32)
    big = jnp.float32(jnp.inf)
    bigf = jnp.float32(float(N))
    dsq = jnp.where(colio == rowio, big, dsq)
    base = b * N
    cols = []
    for _ in range(K):
        m = jnp.min(dsq, axis=1, keepdims=True)
        cand = jnp.where(dsq == m, colf, bigf)
        am = jnp.min(cand, axis=1, keepdims=True)
        cols.append(am)
        dsq = jnp.where(cand == am, big, dsq)
    knn_ref[0] = jnp.concatenate(cols, axis=1).astype(jnp.int32) + base
    g_ref[0] = (
        jnp.dot(x[:, 2:], w1_ref[...], preferred_element_type=jnp.float32)
        + b1_ref[...]
    )


def _knn_call(inputs, pc, w1, b1, half):
    return pl.pallas_call(
        functools.partial(_knn_body, half=half),
        grid=(BH, N // ROWS),
        in_specs=[
            pl.BlockSpec((1, ROWS, 2 + FDIM), lambda b, r: (b + half * BH, r, 0)),
            pl.BlockSpec((1, 2, N), lambda b, r: (b + half * BH, 0, 0)),
            pl.BlockSpec((FDIM, HID), lambda b, r: (0, 0)),
            pl.BlockSpec((1, HID), lambda b, r: (0, 0)),
        ],
        out_specs=[
            pl.BlockSpec((1, ROWS, K), lambda b, r: (b, r, 0)),
            pl.BlockSpec((1, ROWS, HID), lambda b, r: (b, r, 0)),
        ],
        out_shape=[
            jax.ShapeDtypeStruct((BH, N, K), jnp.int32),
            jax.ShapeDtypeStruct((BH, N, HID), jnp.float32),
        ],
    )(inputs, pc, w1, b1)


def _make_gather_mean():
    mesh = plsc.VectorSubcoreMesh(core_axis_name="c", subcore_axis_name="s")

    @functools.partial(
        pl.kernel,
        mesh=mesh,
        compiler_params=pltpu.CompilerParams(use_tc_tiling_on_sc=False),
        out_type=jax.ShapeDtypeStruct((BH * N, HID), jnp.float32),
        scratch_types=[
            pltpu.VMEM((K * CHUNK,), jnp.int32),
            pltpu.VMEM((K * CHUNK,), jnp.int32),
            pltpu.VMEM((K * CHUNK, HID), jnp.float32),
            pltpu.VMEM((K * CHUNK, HID), jnp.float32),
            pltpu.VMEM((CHUNK, HID), jnp.float32),
            pltpu.VMEM((CHUNK, HID), jnp.float32),
            pltpu.SemaphoreType.DMA,
            pltpu.SemaphoreType.DMA,
            pltpu.SemaphoreType.DMA,
            pltpu.SemaphoreType.DMA,
            pltpu.SemaphoreType.DMA,
        ],
    )
    def _gather_mean(idx_hbm, g_hbm, out_hbm, idx0, idx1, rows0, rows1,
                     acc0, acc1, semi0, semi1, semg0, semg1, semo):
        wid = lax.axis_index("s") * NC + lax.axis_index("c")
        idxs = (idx0, idx1)
        rows = (rows0, rows1)
        accs = (acc0, acc1)
        semis = (semi0, semi1)
        semgs = (semg0, semg1)

        def _qbase(c):
            return wid * QPW + c * CHUNK

        idx_cp = [None] * NCHUNK
        for c in range(min(2, NCHUNK)):
            idx_cp[c] = pltpu.async_copy(
                idx_hbm.at[pl.ds(_qbase(c) * K, K * CHUNK)], idxs[c % 2], semis[c % 2]
            )

        def _fire_gathers(c):
            idx_cp[c].wait()
            return [
                pltpu.async_copy(
                    g_hbm.at[idxs[c % 2].at[pl.ds(j * CHUNK, CHUNK)]],
                    rows[c % 2].at[pl.ds(j * CHUNK, CHUNK)],
                    semgs[c % 2],
                )
                for j in range(K)
            ]

        def _accumulate(c):
            def _addq(q):
                r = q * K
                for v in range(HID // 16):
                    sl = pl.ds(v * 16, 16)
                    accs[c % 2][q, sl] = (
                        (rows[c % 2][r, sl] + rows[c % 2][r + 1, sl])
                        + (rows[c % 2][r + 2, sl] + rows[c % 2][r + 3, sl])
                        + (rows[c % 2][r + 4, sl] + rows[c % 2][r + 5, sl])
                    )

            plsc.parallel_loop(0, CHUNK, 1, unroll=4)(_addq)

        g_cp = [None] * NCHUNK
        g_cp[0] = _fire_gathers(0)
        if NCHUNK > 1:
            g_cp[1] = _fire_gathers(1)
        out_cp = [None] * NCHUNK
        for c in range(NCHUNK):
            for cp in g_cp[c]:
                cp.wait()
            if c + 2 < NCHUNK:
                idx_cp[c + 2] = pltpu.async_copy(
                    idx_hbm.at[pl.ds(_qbase(c + 2) * K, K * CHUNK)],
                    idxs[c % 2],
                    semis[c % 2],
                )
            if c >= 2:
                out_cp[c - 2].wait()
            _accumulate(c)
            out_cp[c] = pltpu.async_copy(
                accs[c % 2], out_hbm.at[pl.ds(_qbase(c), CHUNK)], semo
            )
            if c + 2 < NCHUNK:
                g_cp[c + 2] = _fire_gathers(c + 2)
        for c in range(max(0, NCHUNK - 2), NCHUNK):
            out_cp[c].wait()

    return _gather_mean


def _mlp_body(agg_ref, w2_ref, b2_ref, out_ref):
    h = jnp.maximum(agg_ref[...] * jnp.float32(1.0 / 6.0), 0.0)
    out_ref[...] = (
        jnp.dot(h, w2_ref[...], preferred_element_type=jnp.float32) + b2_ref[...]
    )


def _mlp_call(agg, w2, b2):
    return pl.pallas_call(
        _mlp_body,
        grid=((BH * N) // ROWS,),
        in_specs=[
            pl.BlockSpec((ROWS, HID), lambda r: (r, 0)),
            pl.BlockSpec((HID, FDIM), lambda r: (0, 0)),
            pl.BlockSpec((1, FDIM), lambda r: (0, 0)),
        ],
        out_specs=pl.BlockSpec((ROWS, FDIM), lambda r: (r, 0)),
        out_shape=jax.ShapeDtypeStruct((BH * N, FDIM), jnp.float32),
    )(agg, w2, b2)


def kernel(inputs, W1, b1, W2, b2):
    pos = inputs[..., :2]
    pc = jnp.transpose(pos, (0, 2, 1))
    b1r = b1.reshape(1, HID)
    b2r = b2.reshape(1, FDIM)
    gather_mean = _make_gather_mean()
    upds = []
    for half in range(HALVES):
        knn, g = _knn_call(inputs, pc, W1, b1r, half)
        agg = gather_mean(knn.reshape(BH * N * K), g.reshape(BH * N, HID))
        upds.append(_mlp_call(agg, W2, b2r).reshape(BH, N, FDIM))
    upd = jnp.concatenate(upds, axis=0)
    return jnp.concatenate([pos, upd], axis=-1)
